# Initial kernel scaffold; baseline (speedup 1.0000x reference)
import functools

import jax
import jax.numpy as jnp
from jax import lax
from jax.experimental import pallas as pl
from jax.experimental.pallas import tpu as pltpu
from jax.experimental.pallas import tpu_sc as plsc

NC = 2
NS = 16
NW = NC * NS

N = 10000
E = 320000
D = 128
T = 32768
NREL = 100

EPW = E // NW
KB = 80
NB = EPW // KB
RPS = N // NS

TPW = T // NW
TKB = 128
TNB = TPW // TKB

_HIGH = lax.Precision.HIGHEST

_mesh = plsc.VectorSubcoreMesh(core_axis_name="c", subcore_axis_name="s")


def _sc_degree(dst3, ones, zeros16):
    @functools.partial(
        pl.kernel,
        out_type=jax.ShapeDtypeStruct((NC, N, 16), jnp.float32),
        mesh=_mesh,
        scratch_types=[
            pltpu.VMEM((NB, KB), jnp.int32),
            pltpu.VMEM((KB, 16), jnp.float32),
            pltpu.VMEM_SHARED((N, 16), jnp.float32),
        ],
    )
    def k(dst_hbm, ones_hbm, zero_hbm, out_hbm, idx_v, ones_v, acc_sh):
        cid = lax.axis_index("c")
        sid = lax.axis_index("s")
        wid = sid * NC + cid
        stripe = pl.ds(sid * RPS, RPS)
        pltpu.sync_copy(zero_hbm.at[stripe], acc_sh.at[stripe])
        pltpu.sync_copy(ones_hbm, ones_v)
        pltpu.sync_copy(dst_hbm.at[wid], idx_v)
        plsc.subcore_barrier()

        @pl.loop(0, NB)
        def _(j):
            pltpu.sync_copy(ones_v, acc_sh.at[idx_v.at[j]], add=True)

        plsc.subcore_barrier()
        pltpu.sync_copy(acc_sh.at[stripe], out_hbm.at[cid].at[stripe])

    return k(dst3, ones, zeros16)


def _sc_edge_scatter(g, src3, dst3, zeros):
    @functools.partial(
        pl.kernel,
        out_type=jax.ShapeDtypeStruct((NC, N, D), jnp.float32),
        mesh=_mesh,
        scratch_types=[
            pltpu.VMEM((NB, KB), jnp.int32),
            pltpu.VMEM((NB, KB), jnp.int32),
            pltpu.VMEM((KB, D), jnp.float32),
            pltpu.VMEM_SHARED((N, D), jnp.float32),
        ],
    )
    def k(g_hbm, src_hbm, dst_hbm, zero_hbm, out_hbm, src_v, dst_v, rows_v,
          acc_sh):
        cid = lax.axis_index("c")
        sid = lax.axis_index("s")
        wid = sid * NC + cid
        stripe = pl.ds(sid * RPS, RPS)
        pltpu.sync_copy(zero_hbm.at[stripe], acc_sh.at[stripe])
        pltpu.sync_copy(src_hbm.at[wid], src_v)
        pltpu.sync_copy(dst_hbm.at[wid], dst_v)
        plsc.subcore_barrier()

        @pl.loop(0, NB)
        def _(j):
            pltpu.sync_copy(g_hbm.at[src_v.at[j]], rows_v)
            pltpu.sync_copy(rows_v, acc_sh.at[dst_v.at[j]], add=True)

        plsc.subcore_barrier()
        pltpu.sync_copy(acc_sh.at[stripe], out_hbm.at[cid].at[stripe])

    return k(g, src3, dst3, zeros)


def _sc_triple_gather(h, rel_emb, idx3):
    row_t = jax.ShapeDtypeStruct((T, D), jnp.float32)

    @functools.partial(
        pl.kernel,
        out_type=(row_t, row_t, row_t),
        mesh=_mesh,
        scratch_types=[
            pltpu.VMEM((3 * TNB, TKB), jnp.int32),
            pltpu.VMEM((TKB, D), jnp.float32),
        ],
    )
    def k(h_hbm, rel_hbm, idx_hbm, oh_hbm, orel_hbm, ot_hbm, idx_v, rows_v):
        cid = lax.axis_index("c")
        sid = lax.axis_index("s")
        wid = sid * NC + cid
        base = wid * TPW
        pltpu.sync_copy(idx_hbm.at[wid], idx_v)

        @pl.loop(0, TNB)
        def _(j):
            dst = pl.ds(base + j * TKB, TKB)
            pltpu.sync_copy(h_hbm.at[idx_v.at[j]], rows_v)
            pltpu.sync_copy(rows_v, oh_hbm.at[dst])
            pltpu.sync_copy(rel_hbm.at[idx_v.at[TNB + j]], rows_v)
            pltpu.sync_copy(rows_v, orel_hbm.at[dst])
            pltpu.sync_copy(h_hbm.at[idx_v.at[2 * TNB + j]], rows_v)
            pltpu.sync_copy(rows_v, ot_hbm.at[dst])

    return k(h, rel_emb, idx3)


def _dinv_from(degp_a, degp_b):
    return lax.rsqrt(degp_a[:, 0:1] + degp_b[:, 0:1] + 1.0)


def _tc_pre1(x, W1, degp):
    def body(x_ref, w_ref, deg_ref, o_ref):
        dinv = _dinv_from(deg_ref[0], deg_ref[1])
        h = jnp.dot(x_ref[...], w_ref[...],
                    preferred_element_type=jnp.float32, precision=_HIGH)
        o_ref[...] = h * dinv

    return pl.pallas_call(
        body, out_shape=jax.ShapeDtypeStruct((N, D), jnp.float32),
    )(x, W1, degp)


def _tc_mid(accp, g1, degp, b1, W2):
    def body(acc_ref, g_ref, deg_ref, b_ref, w_ref, o_ref):
        dinv = _dinv_from(deg_ref[0], deg_ref[1])
        s = acc_ref[0] + acc_ref[1] + g_ref[...]
        a1 = jnp.maximum(s * dinv + b_ref[...], 0.0)
        h2 = jnp.dot(a1, w_ref[...],
                     preferred_element_type=jnp.float32, precision=_HIGH)
        o_ref[...] = h2 * dinv

    return pl.pallas_call(
        body, out_shape=jax.ShapeDtypeStruct((N, D), jnp.float32),
    )(accp, g1, degp, b1, W2)


def _tc_post2(accp, g2, degp, b2):
    def body(acc_ref, g_ref, deg_ref, b_ref, o_ref):
        dinv = _dinv_from(deg_ref[0], deg_ref[1])
        s = acc_ref[0] + acc_ref[1] + g_ref[...]
        o_ref[...] = jnp.maximum(s * dinv + b_ref[...], 0.0)

    return pl.pallas_call(
        body, out_shape=jax.ShapeDtypeStruct((N, D), jnp.float32),
    )(accp, g2, degp, b2)


_TB = 4096


def _tc_mlp(hh, hr, ht, Wm1, bm1, Wm2p, bm2p):
    def body(hh_ref, hr_ref, ht_ref, w1_ref, b1_ref, w2_ref, b2_ref, o_ref):
        t = hh_ref[...] + hr_ref[...] + ht_ref[...]
        q = jnp.maximum(
            jnp.dot(t, w1_ref[...], preferred_element_type=jnp.float32,
                    precision=_HIGH) + b1_ref[...], 0.0)
        o_ref[...] = jnp.dot(q, w2_ref[...],
                             preferred_element_type=jnp.float32,
                             precision=_HIGH) + b2_ref[...]

    row_spec = pl.BlockSpec((_TB, D), lambda i: (i, 0))
    full = pl.BlockSpec((D, D), lambda i: (0, 0))
    vec = pl.BlockSpec((1, D), lambda i: (0, 0))
    return pl.pallas_call(
        body,
        grid=(T // _TB,),
        in_specs=[row_spec, row_spec, row_spec, full, vec, full, vec],
        out_specs=row_spec,
        out_shape=jax.ShapeDtypeStruct((T, D), jnp.float32),
    )(hh, hr, ht, Wm1, bm1, Wm2p, bm2p)


def kernel(x, edge_index, head_idx, tail_idx, rel_idx, W1, b1, W2, b2,
           rel_emb, Wm1, bm1, Wm2, bm2):
    src3 = edge_index[0].reshape(NW, NB, KB)
    dst3 = edge_index[1].reshape(NW, NB, KB)

    ones = jnp.ones((KB, 16), jnp.float32)
    zeros16 = jnp.zeros((N, 16), jnp.float32)
    zeros = jnp.zeros((N, D), jnp.float32)

    hh3 = head_idx.reshape(NW, TNB, TKB)
    rr3 = rel_idx.reshape(NW, TNB, TKB)
    tt3 = tail_idx.reshape(NW, TNB, TKB)
    idx3 = jnp.concatenate([hh3, rr3, tt3], axis=1)

    degp = _sc_degree(dst3, ones, zeros16)

    g1 = _tc_pre1(x, W1, degp)
    acc1 = _sc_edge_scatter(g1, src3, dst3, zeros)
    g2 = _tc_mid(acc1, g1, degp, b1.reshape(1, D), W2)
    acc2 = _sc_edge_scatter(g2, src3, dst3, zeros)
    h = _tc_post2(acc2, g2, degp, b2.reshape(1, D))

    hh, hr, ht = _sc_triple_gather(h, rel_emb, idx3)

    Wm2p = jnp.zeros((D, D), jnp.float32).at[:, :3].set(Wm2)
    bm2p = jnp.zeros((1, D), jnp.float32).at[0, :3].set(bm2)
    out = _tc_mlp(hh, hr, ht, Wm1, bm1.reshape(1, D), Wm2p, bm2p)
    return out[:, :3]


# speedup vs baseline: 14.3031x; 14.3031x over previous
import functools

import jax
import jax.numpy as jnp
from jax import lax
from jax.experimental import pallas as pl
from jax.experimental.pallas import tpu as pltpu
from jax.experimental.pallas import tpu_sc as plsc

NC = 2
NS = 16
NW = NC * NS

N = 10000
E = 320000
D = 128
T = 32768
NREL = 100

EPW = E // NW
KB = 80
NB = EPW // KB
S_LO = 624
S_HI = 640

TPW = T // NW
TKB = 128
TNB = TPW // TKB

_HIGH = lax.Precision.HIGHEST

_mesh = plsc.VectorSubcoreMesh(core_axis_name="c", subcore_axis_name="s")


def _stripe_copy(sid, refs_fn):
    @pl.when(sid < NS - 1)
    def _():
        src, dst = refs_fn(pl.ds(sid * S_LO, S_LO))
        pltpu.sync_copy(src, dst)

    @pl.when(sid == NS - 1)
    def _():
        src, dst = refs_fn(pl.ds((NS - 1) * S_LO, S_HI))
        pltpu.sync_copy(src, dst)


def _sc_degree(dst3, ones, zeros16):
    @functools.partial(
        pl.kernel,
        out_type=jax.ShapeDtypeStruct((NC, N, D), jnp.float32),
        mesh=_mesh,
        scratch_types=[
            pltpu.VMEM((NB, KB), jnp.int32),
            pltpu.VMEM((KB, D), jnp.float32),
            pltpu.VMEM_SHARED((N, D), jnp.float32),
        ],
    )
    def k(dst_hbm, ones_hbm, zero_hbm, out_hbm, idx_v, ones_v, acc_sh):
        cid = lax.axis_index("c")
        sid = lax.axis_index("s")
        wid = sid * NC + cid
        _stripe_copy(sid, lambda s: (zero_hbm.at[s], acc_sh.at[s]))
        pltpu.sync_copy(ones_hbm, ones_v)
        pltpu.sync_copy(dst_hbm.at[wid], idx_v)
        plsc.subcore_barrier()

        @pl.loop(0, NB)
        def _(j):
            pltpu.sync_copy(ones_v, acc_sh.at[idx_v.at[j]], add=True)

        plsc.subcore_barrier()
        _stripe_copy(sid, lambda s: (acc_sh.at[s], out_hbm.at[cid].at[s]))

    return k(dst3, ones, zeros16)


def _sc_edge_scatter(g, src3, dst3, zeros):
    @functools.partial(
        pl.kernel,
        out_type=jax.ShapeDtypeStruct((NC, N, D), jnp.float32),
        mesh=_mesh,
        scratch_types=[
            pltpu.VMEM((NB, KB), jnp.int32),
            pltpu.VMEM((NB, KB), jnp.int32),
            pltpu.VMEM((KB, D), jnp.float32),
            pltpu.VMEM_SHARED((N, D), jnp.float32),
        ],
    )
    def k(g_hbm, src_hbm, dst_hbm, zero_hbm, out_hbm, src_v, dst_v, rows_v,
          acc_sh):
        cid = lax.axis_index("c")
        sid = lax.axis_index("s")
        wid = sid * NC + cid
        _stripe_copy(sid, lambda s: (zero_hbm.at[s], acc_sh.at[s]))
        pltpu.sync_copy(src_hbm.at[wid], src_v)
        pltpu.sync_copy(dst_hbm.at[wid], dst_v)
        plsc.subcore_barrier()

        @pl.loop(0, NB)
        def _(j):
            pltpu.sync_copy(g_hbm.at[src_v.at[j]], rows_v)
            pltpu.sync_copy(rows_v, acc_sh.at[dst_v.at[j]], add=True)

        plsc.subcore_barrier()
        _stripe_copy(sid, lambda s: (acc_sh.at[s], out_hbm.at[cid].at[s]))

    return k(g, src3, dst3, zeros)


def _sc_triple_gather(h, rel_emb, idx3):
    row_t = jax.ShapeDtypeStruct((T, D), jnp.float32)

    @functools.partial(
        pl.kernel,
        out_type=(row_t, row_t, row_t),
        mesh=_mesh,
        scratch_types=[
            pltpu.VMEM((3 * TNB, TKB), jnp.int32),
            pltpu.VMEM((TKB, D), jnp.float32),
        ],
    )
    def k(h_hbm, rel_hbm, idx_hbm, oh_hbm, orel_hbm, ot_hbm, idx_v, rows_v):
        cid = lax.axis_index("c")
        sid = lax.axis_index("s")
        wid = sid * NC + cid
        base = wid * TPW
        pltpu.sync_copy(idx_hbm.at[wid], idx_v)

        @pl.loop(0, TNB)
        def _(j):
            dst = pl.ds(base + j * TKB, TKB)
            pltpu.sync_copy(h_hbm.at[idx_v.at[j]], rows_v)
            pltpu.sync_copy(rows_v, oh_hbm.at[dst])
            pltpu.sync_copy(rel_hbm.at[idx_v.at[TNB + j]], rows_v)
            pltpu.sync_copy(rows_v, orel_hbm.at[dst])
            pltpu.sync_copy(h_hbm.at[idx_v.at[2 * TNB + j]], rows_v)
            pltpu.sync_copy(rows_v, ot_hbm.at[dst])

    return k(h, rel_emb, idx3)


def _dinv_from(degp_a, degp_b):
    return lax.rsqrt(degp_a[:, 0:1] + degp_b[:, 0:1] + 1.0)


def _tc_pre1(x, W1, degp):
    def body(x_ref, w_ref, deg_ref, o_ref):
        dinv = _dinv_from(deg_ref[0], deg_ref[1])
        h = jnp.dot(x_ref[...], w_ref[...],
                    preferred_element_type=jnp.float32, precision=_HIGH)
        o_ref[...] = h * dinv

    return pl.pallas_call(
        body, out_shape=jax.ShapeDtypeStruct((N, D), jnp.float32),
    )(x, W1, degp)


def _tc_mid(accp, g1, degp, b1, W2):
    def body(acc_ref, g_ref, deg_ref, b_ref, w_ref, o_ref):
        dinv = _dinv_from(deg_ref[0], deg_ref[1])
        s = acc_ref[0] + acc_ref[1] + g_ref[...]
        a1 = jnp.maximum(s * dinv + b_ref[...], 0.0)
        h2 = jnp.dot(a1, w_ref[...],
                     preferred_element_type=jnp.float32, precision=_HIGH)
        o_ref[...] = h2 * dinv

    return pl.pallas_call(
        body, out_shape=jax.ShapeDtypeStruct((N, D), jnp.float32),
    )(accp, g1, degp, b1, W2)


def _tc_post2(accp, g2, degp, b2):
    def body(acc_ref, g_ref, deg_ref, b_ref, o_ref):
        dinv = _dinv_from(deg_ref[0], deg_ref[1])
        s = acc_ref[0] + acc_ref[1] + g_ref[...]
        o_ref[...] = jnp.maximum(s * dinv + b_ref[...], 0.0)

    return pl.pallas_call(
        body, out_shape=jax.ShapeDtypeStruct((N, D), jnp.float32),
    )(accp, g2, degp, b2)


_TB = 4096


def _tc_mlp(hh, hr, ht, Wm1, bm1, Wm2p, bm2p):
    def body(hh_ref, hr_ref, ht_ref, w1_ref, b1_ref, w2_ref, b2_ref, o_ref):
        t = hh_ref[...] + hr_ref[...] + ht_ref[...]
        q = jnp.maximum(
            jnp.dot(t, w1_ref[...], preferred_element_type=jnp.float32,
                    precision=_HIGH) + b1_ref[...], 0.0)
        o_ref[...] = jnp.dot(q, w2_ref[...],
                             preferred_element_type=jnp.float32,
                             precision=_HIGH) + b2_ref[...]

    row_spec = pl.BlockSpec((_TB, D), lambda i: (i, 0))
    full = pl.BlockSpec((D, D), lambda i: (0, 0))
    vec = pl.BlockSpec((1, D), lambda i: (0, 0))
    return pl.pallas_call(
        body,
        grid=(T // _TB,),
        in_specs=[row_spec, row_spec, row_spec, full, vec, full, vec],
        out_specs=row_spec,
        out_shape=jax.ShapeDtypeStruct((T, D), jnp.float32),
    )(hh, hr, ht, Wm1, bm1, Wm2p, bm2p)


def kernel(x, edge_index, head_idx, tail_idx, rel_idx, W1, b1, W2, b2,
           rel_emb, Wm1, bm1, Wm2, bm2):
    src3 = edge_index[0].reshape(NW, NB, KB)
    dst3 = edge_index[1].reshape(NW, NB, KB)

    ones = jnp.ones((KB, D), jnp.float32)
    zeros = jnp.zeros((N, D), jnp.float32)

    hh3 = head_idx.reshape(NW, TNB, TKB)
    rr3 = rel_idx.reshape(NW, TNB, TKB)
    tt3 = tail_idx.reshape(NW, TNB, TKB)
    idx3 = jnp.concatenate([hh3, rr3, tt3], axis=1)

    degp = _sc_degree(dst3, ones, zeros)

    g1 = _tc_pre1(x, W1, degp)
    acc1 = _sc_edge_scatter(g1, src3, dst3, zeros)
    g2 = _tc_mid(acc1, g1, degp, b1.reshape(1, D), W2)
    acc2 = _sc_edge_scatter(g2, src3, dst3, zeros)
    h = _tc_post2(acc2, g2, degp, b2.reshape(1, D))

    hh, hr, ht = _sc_triple_gather(h, rel_emb, idx3)

    Wm2p = jnp.zeros((D, D), jnp.float32).at[:, :3].set(Wm2)
    bm2p = jnp.zeros((1, D), jnp.float32).at[0, :3].set(bm2)
    out = _tc_mlp(hh, hr, ht, Wm1, bm1.reshape(1, D), Wm2p, bm2p)
    return out[:, :3]


# speedup vs baseline: 18.3903x; 1.2858x over previous
import functools

import jax
import jax.numpy as jnp
from jax import lax
from jax.experimental import pallas as pl
from jax.experimental.pallas import tpu as pltpu
from jax.experimental.pallas import tpu_sc as plsc

NC = 2
NS = 16
NW = NC * NS

N = 10000
E = 320000
D = 128
T = 32768
NREL = 100

EPW = E // NW
KB = 80
NB = EPW // KB
CNB = 25
NCH = NB // CNB
S_LO = 624
S_HI = 640

TPW = T // NW
TKB = 128
TNB = TPW // TKB

_HIGH = lax.Precision.HIGHEST

_mesh = plsc.VectorSubcoreMesh(core_axis_name="c", subcore_axis_name="s")


def _stripe_copy(sid, refs_fn):
    @pl.when(sid < NS - 1)
    def _():
        src, dst = refs_fn(pl.ds(sid * S_LO, S_LO))
        pltpu.sync_copy(src, dst)

    @pl.when(sid == NS - 1)
    def _():
        src, dst = refs_fn(pl.ds((NS - 1) * S_LO, S_HI))
        pltpu.sync_copy(src, dst)


def _sc_degree(dst3, ones, zeros16):
    @functools.partial(
        pl.kernel,
        out_type=jax.ShapeDtypeStruct((NC, N, D), jnp.float32),
        mesh=_mesh,
        scratch_types=[
            pltpu.VMEM((NB, KB), jnp.int32),
            pltpu.VMEM((KB, D), jnp.float32),
            pltpu.VMEM_SHARED((N, D), jnp.float32),
        ],
    )
    def k(dst_hbm, ones_hbm, zero_hbm, out_hbm, idx_v, ones_v, acc_sh):
        cid = lax.axis_index("c")
        sid = lax.axis_index("s")
        wid = sid * NC + cid
        _stripe_copy(sid, lambda s: (zero_hbm.at[s], acc_sh.at[s]))
        pltpu.sync_copy(ones_hbm, ones_v)
        pltpu.sync_copy(dst_hbm.at[wid], idx_v)
        plsc.subcore_barrier()

        @pl.loop(0, NB)
        def _(j):
            pltpu.sync_copy(ones_v, acc_sh.at[idx_v.at[j]], add=True)

        plsc.subcore_barrier()
        _stripe_copy(sid, lambda s: (acc_sh.at[s], out_hbm.at[cid].at[s]))

    return k(dst3, ones, zeros16)


def _sc_edge_scatter(g, src4, dst4, zeros):
    @functools.partial(
        pl.kernel,
        out_type=jax.ShapeDtypeStruct((NC, N, D), jnp.float32),
        mesh=_mesh,
        scratch_types=[
            pltpu.VMEM((CNB, KB), jnp.int32),
            pltpu.VMEM((CNB, KB), jnp.int32),
            pltpu.VMEM((KB, D), jnp.float32),
            pltpu.VMEM((KB, D), jnp.float32),
            pltpu.VMEM_SHARED((N, D), jnp.float32),
            pltpu.SemaphoreType.DMA,
            pltpu.SemaphoreType.DMA,
        ],
    )
    def k(g_hbm, src_hbm, dst_hbm, zero_hbm, out_hbm, src_v, dst_v, rows_a,
          rows_b, acc_sh, sem_a, sem_b):
        cid = lax.axis_index("c")
        sid = lax.axis_index("s")
        wid = sid * NC + cid
        _stripe_copy(sid, lambda s: (zero_hbm.at[s], acc_sh.at[s]))
        plsc.subcore_barrier()

        @pl.loop(0, NCH)
        def _(c):
            pltpu.sync_copy(src_hbm.at[wid].at[c], src_v)
            pltpu.sync_copy(dst_hbm.at[wid].at[c], dst_v)
            pltpu.async_copy(g_hbm.at[src_v.at[0]], rows_a, sem_a)

            @pl.loop(0, (CNB - 1) // 2)
            def _(i):
                j = 2 * i
                pltpu.async_copy(g_hbm.at[src_v.at[j + 1]], rows_b, sem_b)
                pltpu.make_async_copy(
                    g_hbm.at[src_v.at[j]], rows_a, sem_a).wait()
                pltpu.sync_copy(rows_a, acc_sh.at[dst_v.at[j]], add=True)
                pltpu.async_copy(g_hbm.at[src_v.at[j + 2]], rows_a, sem_a)
                pltpu.make_async_copy(
                    g_hbm.at[src_v.at[j + 1]], rows_b, sem_b).wait()
                pltpu.sync_copy(rows_b, acc_sh.at[dst_v.at[j + 1]], add=True)

            pltpu.make_async_copy(
                g_hbm.at[src_v.at[CNB - 1]], rows_a, sem_a).wait()
            pltpu.sync_copy(rows_a, acc_sh.at[dst_v.at[CNB - 1]], add=True)

        plsc.subcore_barrier()
        _stripe_copy(sid, lambda s: (acc_sh.at[s], out_hbm.at[cid].at[s]))

    return k(g, src4, dst4, zeros)


def _sc_triple_gather(h, rel_emb, idx3):
    row_t = jax.ShapeDtypeStruct((T, D), jnp.float32)

    @functools.partial(
        pl.kernel,
        out_type=(row_t, row_t, row_t),
        mesh=_mesh,
        scratch_types=[
            pltpu.VMEM((3 * TNB, TKB), jnp.int32),
            pltpu.VMEM((TKB, D), jnp.float32),
        ],
    )
    def k(h_hbm, rel_hbm, idx_hbm, oh_hbm, orel_hbm, ot_hbm, idx_v, rows_v):
        cid = lax.axis_index("c")
        sid = lax.axis_index("s")
        wid = sid * NC + cid
        base = wid * TPW
        pltpu.sync_copy(idx_hbm.at[wid], idx_v)

        @pl.loop(0, TNB)
        def _(j):
            dst = pl.ds(base + j * TKB, TKB)
            pltpu.sync_copy(h_hbm.at[idx_v.at[j]], rows_v)
            pltpu.sync_copy(rows_v, oh_hbm.at[dst])
            pltpu.sync_copy(rel_hbm.at[idx_v.at[TNB + j]], rows_v)
            pltpu.sync_copy(rows_v, orel_hbm.at[dst])
            pltpu.sync_copy(h_hbm.at[idx_v.at[2 * TNB + j]], rows_v)
            pltpu.sync_copy(rows_v, ot_hbm.at[dst])

    return k(h, rel_emb, idx3)


def _dinv_from(degp_a, degp_b):
    return lax.rsqrt(degp_a[:, 0:1] + degp_b[:, 0:1] + 1.0)


def _tc_pre1(x, W1, degp):
    def body(x_ref, w_ref, deg_ref, o_ref):
        dinv = _dinv_from(deg_ref[0], deg_ref[1])
        h = jnp.dot(x_ref[...], w_ref[...],
                    preferred_element_type=jnp.float32, precision=_HIGH)
        o_ref[...] = h * dinv

    return pl.pallas_call(
        body, out_shape=jax.ShapeDtypeStruct((N, D), jnp.float32),
    )(x, W1, degp)


def _tc_mid(accp, g1, degp, b1, W2):
    def body(acc_ref, g_ref, deg_ref, b_ref, w_ref, o_ref):
        dinv = _dinv_from(deg_ref[0], deg_ref[1])
        s = acc_ref[0] + acc_ref[1] + g_ref[...]
        a1 = jnp.maximum(s * dinv + b_ref[...], 0.0)
        h2 = jnp.dot(a1, w_ref[...],
                     preferred_element_type=jnp.float32, precision=_HIGH)
        o_ref[...] = h2 * dinv

    return pl.pallas_call(
        body, out_shape=jax.ShapeDtypeStruct((N, D), jnp.float32),
    )(accp, g1, degp, b1, W2)


def _tc_post2(accp, g2, degp, b2):
    def body(acc_ref, g_ref, deg_ref, b_ref, o_ref):
        dinv = _dinv_from(deg_ref[0], deg_ref[1])
        s = acc_ref[0] + acc_ref[1] + g_ref[...]
        o_ref[...] = jnp.maximum(s * dinv + b_ref[...], 0.0)

    return pl.pallas_call(
        body, out_shape=jax.ShapeDtypeStruct((N, D), jnp.float32),
    )(accp, g2, degp, b2)


_TB = 4096


def _tc_mlp(hh, hr, ht, Wm1, bm1, Wm2p, bm2p):
    def body(hh_ref, hr_ref, ht_ref, w1_ref, b1_ref, w2_ref, b2_ref, o_ref):
        t = hh_ref[...] + hr_ref[...] + ht_ref[...]
        q = jnp.maximum(
            jnp.dot(t, w1_ref[...], preferred_element_type=jnp.float32,
                    precision=_HIGH) + b1_ref[...], 0.0)
        o_ref[...] = jnp.dot(q, w2_ref[...],
                             preferred_element_type=jnp.float32,
                             precision=_HIGH) + b2_ref[...]

    row_spec = pl.BlockSpec((_TB, D), lambda i: (i, 0))
    full = pl.BlockSpec((D, D), lambda i: (0, 0))
    vec = pl.BlockSpec((1, D), lambda i: (0, 0))
    return pl.pallas_call(
        body,
        grid=(T // _TB,),
        in_specs=[row_spec, row_spec, row_spec, full, vec, full, vec],
        out_specs=row_spec,
        out_shape=jax.ShapeDtypeStruct((T, D), jnp.float32),
    )(hh, hr, ht, Wm1, bm1, Wm2p, bm2p)


def kernel(x, edge_index, head_idx, tail_idx, rel_idx, W1, b1, W2, b2,
           rel_emb, Wm1, bm1, Wm2, bm2):
    src3 = edge_index[0].reshape(NW, NB, KB)
    dst3 = edge_index[1].reshape(NW, NB, KB)
    src4 = src3.reshape(NW, NCH, CNB, KB)
    dst4 = dst3.reshape(NW, NCH, CNB, KB)

    ones = jnp.ones((KB, D), jnp.float32)
    zeros = jnp.zeros((N, D), jnp.float32)

    hh3 = head_idx.reshape(NW, TNB, TKB)
    rr3 = rel_idx.reshape(NW, TNB, TKB)
    tt3 = tail_idx.reshape(NW, TNB, TKB)
    idx3 = jnp.concatenate([hh3, rr3, tt3], axis=1)

    degp = _sc_degree(dst3, ones, zeros)

    g1 = _tc_pre1(x, W1, degp)
    acc1 = _sc_edge_scatter(g1, src4, dst4, zeros)
    g2 = _tc_mid(acc1, g1, degp, b1.reshape(1, D), W2)
    acc2 = _sc_edge_scatter(g2, src4, dst4, zeros)
    h = _tc_post2(acc2, g2, degp, b2.reshape(1, D))

    hh, hr, ht = _sc_triple_gather(h, rel_emb, idx3)

    Wm2p = jnp.zeros((D, D), jnp.float32).at[:, :3].set(Wm2)
    bm2p = jnp.zeros((1, D), jnp.float32).at[0, :3].set(bm2)
    out = _tc_mlp(hh, hr, ht, Wm1, bm1.reshape(1, D), Wm2p, bm2p)
    return out[:, :3]


# speedup vs baseline: 20.0321x; 1.0893x over previous
import functools

import jax
import jax.numpy as jnp
from jax import lax
from jax.experimental import pallas as pl
from jax.experimental.pallas import tpu as pltpu
from jax.experimental.pallas import tpu_sc as plsc

NC = 2
NS = 16
NW = NC * NS

N = 10000
E = 320000
D = 128
T = 32768
NREL = 100

EPW = E // NW
KB = 80
NB = EPW // KB
CNB = 25
NCH = NB // CNB
S_LO = 624
S_HI = 640

TPW = T // NW
TKB = 128
TNB = TPW // TKB

_HIGH = lax.Precision.HIGHEST

_mesh = plsc.VectorSubcoreMesh(core_axis_name="c", subcore_axis_name="s")


def _stripe_copy(sid, refs_fn):
    @pl.when(sid < NS - 1)
    def _():
        src, dst = refs_fn(pl.ds(sid * S_LO, S_LO))
        pltpu.sync_copy(src, dst)

    @pl.when(sid == NS - 1)
    def _():
        src, dst = refs_fn(pl.ds((NS - 1) * S_LO, S_HI))
        pltpu.sync_copy(src, dst)


def _sc_degree(dst3, ones, zeros16):
    @functools.partial(
        pl.kernel,
        out_type=jax.ShapeDtypeStruct((NC, N, D), jnp.float32),
        mesh=_mesh,
        scratch_types=[
            pltpu.VMEM((NB, KB), jnp.int32),
            pltpu.VMEM((KB, D), jnp.float32),
            pltpu.VMEM_SHARED((N, D), jnp.float32),
            pltpu.SemaphoreType.DMA,
        ],
    )
    def k(dst_hbm, ones_hbm, zero_hbm, out_hbm, idx_v, ones_v, acc_sh, sem):
        cid = lax.axis_index("c")
        sid = lax.axis_index("s")
        wid = sid * NC + cid
        _stripe_copy(sid, lambda s: (zero_hbm.at[s], acc_sh.at[s]))
        pltpu.sync_copy(ones_hbm, ones_v)
        pltpu.sync_copy(dst_hbm.at[wid], idx_v)
        plsc.subcore_barrier()

        @pl.loop(0, NB)
        def _(j):
            pltpu.async_copy(ones_v, acc_sh.at[idx_v.at[j]], sem, add=True)

        @pl.loop(0, NB)
        def _(j):
            pltpu.make_async_copy(ones_v, acc_sh.at[idx_v.at[j]], sem).wait()

        plsc.subcore_barrier()
        _stripe_copy(sid, lambda s: (acc_sh.at[s], out_hbm.at[cid].at[s]))

    return k(dst3, ones, zeros16)


def _sc_edge_scatter(g, src4, dst4, zeros):
    @functools.partial(
        pl.kernel,
        out_type=jax.ShapeDtypeStruct((NC, N, D), jnp.float32),
        mesh=_mesh,
        scratch_types=[
            pltpu.VMEM((CNB, KB), jnp.int32),
            pltpu.VMEM((CNB, KB), jnp.int32),
            pltpu.VMEM((KB, D), jnp.float32),
            pltpu.VMEM((KB, D), jnp.float32),
            pltpu.VMEM((KB, D), jnp.float32),
            pltpu.VMEM_SHARED((N, D), jnp.float32),
            pltpu.SemaphoreType.DMA,
            pltpu.SemaphoreType.DMA,
            pltpu.SemaphoreType.DMA,
        ],
    )
    def k(g_hbm, src_hbm, dst_hbm, zero_hbm, out_hbm, src_v, dst_v, rows_a,
          rows_b, rows_c, acc_sh, sem_a, sem_b, sem_c):
        cid = lax.axis_index("c")
        sid = lax.axis_index("s")
        wid = sid * NC + cid
        _stripe_copy(sid, lambda s: (zero_hbm.at[s], acc_sh.at[s]))
        plsc.subcore_barrier()

        bufs = ((rows_a, sem_a), (rows_b, sem_b), (rows_c, sem_c))

        @pl.loop(0, NCH)
        def _(c):
            pltpu.sync_copy(src_hbm.at[wid].at[c], src_v)
            pltpu.sync_copy(dst_hbm.at[wid].at[c], dst_v)
            pltpu.async_copy(g_hbm.at[src_v.at[0]], rows_a, sem_a)
            pltpu.async_copy(g_hbm.at[src_v.at[1]], rows_b, sem_b)
            pltpu.async_copy(g_hbm.at[src_v.at[2]], rows_c, sem_c)

            @pl.loop(0, CNB // 3)
            def _(i):
                j3 = 3 * i
                for off, (rows, sem) in enumerate(bufs):
                    j = j3 + off
                    pltpu.make_async_copy(
                        g_hbm.at[src_v.at[j]], rows, sem).wait()
                    pltpu.sync_copy(rows, acc_sh.at[dst_v.at[j]], add=True)

                    @pl.when(j + 3 < CNB)
                    def _():
                        pltpu.async_copy(
                            g_hbm.at[src_v.at[j + 3]], rows, sem)

            pltpu.make_async_copy(
                g_hbm.at[src_v.at[CNB - 1]], rows_a, sem_a).wait()
            pltpu.sync_copy(rows_a, acc_sh.at[dst_v.at[CNB - 1]], add=True)

        plsc.subcore_barrier()
        _stripe_copy(sid, lambda s: (acc_sh.at[s], out_hbm.at[cid].at[s]))

    return k(g, src4, dst4, zeros)


def _sc_triple_gather(h, rel_emb, idx3):
    row_t = jax.ShapeDtypeStruct((T, D), jnp.float32)

    @functools.partial(
        pl.kernel,
        out_type=(row_t, row_t, row_t),
        mesh=_mesh,
        scratch_types=[
            pltpu.VMEM((3 * TNB, TKB), jnp.int32),
            pltpu.VMEM((3, TKB, D), jnp.float32),
            pltpu.VMEM((3, TKB, D), jnp.float32),
            pltpu.SemaphoreType.DMA,
            pltpu.SemaphoreType.DMA,
        ],
    )
    def k(h_hbm, rel_hbm, idx_hbm, oh_hbm, orel_hbm, ot_hbm, idx_v, buf_a,
          buf_b, sem_a, sem_b):
        cid = lax.axis_index("c")
        sid = lax.axis_index("s")
        wid = sid * NC + cid
        base = wid * TPW
        pltpu.sync_copy(idx_hbm.at[wid], idx_v)

        def fire(buf, sem, j):
            pltpu.async_copy(h_hbm.at[idx_v.at[j]], buf.at[0], sem)
            pltpu.async_copy(rel_hbm.at[idx_v.at[TNB + j]], buf.at[1], sem)
            pltpu.async_copy(h_hbm.at[idx_v.at[2 * TNB + j]], buf.at[2], sem)

        def drain_and_write(buf, sem, j):
            pltpu.make_async_copy(h_hbm.at[idx_v.at[j]], buf.at[0],
                                  sem).wait()
            pltpu.make_async_copy(rel_hbm.at[idx_v.at[TNB + j]], buf.at[1],
                                  sem).wait()
            pltpu.make_async_copy(h_hbm.at[idx_v.at[2 * TNB + j]], buf.at[2],
                                  sem).wait()
            dst = pl.ds(base + j * TKB, TKB)
            pltpu.sync_copy(buf.at[0], oh_hbm.at[dst])
            pltpu.sync_copy(buf.at[1], orel_hbm.at[dst])
            pltpu.sync_copy(buf.at[2], ot_hbm.at[dst])

        fire(buf_a, sem_a, 0)

        @pl.loop(0, TNB // 2)
        def _(i):
            j = 2 * i
            fire(buf_b, sem_b, j + 1)
            drain_and_write(buf_a, sem_a, j)

            @pl.when(j + 2 < TNB)
            def _():
                fire(buf_a, sem_a, j + 2)

            drain_and_write(buf_b, sem_b, j + 1)

    return k(h, rel_emb, idx3)


def _dinv_from(degp_a, degp_b):
    return lax.rsqrt(degp_a[:, 0:1] + degp_b[:, 0:1] + 1.0)


def _tc_mm1(x, W1):
    def body(x_ref, w_ref, o_ref):
        o_ref[...] = jnp.dot(x_ref[...], w_ref[...],
                             preferred_element_type=jnp.float32,
                             precision=_HIGH)

    return pl.pallas_call(
        body, out_shape=jax.ShapeDtypeStruct((N, D), jnp.float32),
    )(x, W1)


def _tc_scale(h1, degp):
    def body(h_ref, deg_ref, o_ref):
        dinv = _dinv_from(deg_ref[0], deg_ref[1])
        o_ref[...] = h_ref[...] * dinv

    return pl.pallas_call(
        body, out_shape=jax.ShapeDtypeStruct((N, D), jnp.float32),
    )(h1, degp)


def _tc_mid(accp, g1, degp, b1, W2):
    def body(acc_ref, g_ref, deg_ref, b_ref, w_ref, o_ref):
        dinv = _dinv_from(deg_ref[0], deg_ref[1])
        s = acc_ref[0] + acc_ref[1] + g_ref[...]
        a1 = jnp.maximum(s * dinv + b_ref[...], 0.0)
        h2 = jnp.dot(a1, w_ref[...],
                     preferred_element_type=jnp.float32, precision=_HIGH)
        o_ref[...] = h2 * dinv

    return pl.pallas_call(
        body, out_shape=jax.ShapeDtypeStruct((N, D), jnp.float32),
    )(accp, g1, degp, b1, W2)


def _tc_post2(accp, g2, degp, b2):
    def body(acc_ref, g_ref, deg_ref, b_ref, o_ref):
        dinv = _dinv_from(deg_ref[0], deg_ref[1])
        s = acc_ref[0] + acc_ref[1] + g_ref[...]
        o_ref[...] = jnp.maximum(s * dinv + b_ref[...], 0.0)

    return pl.pallas_call(
        body, out_shape=jax.ShapeDtypeStruct((N, D), jnp.float32),
    )(accp, g2, degp, b2)


_TB = 4096


def _tc_mlp(hh, hr, ht, Wm1, bm1, Wm2p, bm2p):
    def body(hh_ref, hr_ref, ht_ref, w1_ref, b1_ref, w2_ref, b2_ref, o_ref):
        t = hh_ref[...] + hr_ref[...] + ht_ref[...]
        q = jnp.maximum(
            jnp.dot(t, w1_ref[...], preferred_element_type=jnp.float32,
                    precision=_HIGH) + b1_ref[...], 0.0)
        o_ref[...] = jnp.dot(q, w2_ref[...],
                             preferred_element_type=jnp.float32,
                             precision=_HIGH) + b2_ref[...]

    row_spec = pl.BlockSpec((_TB, D), lambda i: (i, 0))
    full = pl.BlockSpec((D, D), lambda i: (0, 0))
    vec = pl.BlockSpec((1, D), lambda i: (0, 0))
    return pl.pallas_call(
        body,
        grid=(T // _TB,),
        in_specs=[row_spec, row_spec, row_spec, full, vec, full, vec],
        out_specs=row_spec,
        out_shape=jax.ShapeDtypeStruct((T, D), jnp.float32),
    )(hh, hr, ht, Wm1, bm1, Wm2p, bm2p)


def kernel(x, edge_index, head_idx, tail_idx, rel_idx, W1, b1, W2, b2,
           rel_emb, Wm1, bm1, Wm2, bm2):
    src3 = edge_index[0].reshape(NW, NB, KB)
    dst3 = edge_index[1].reshape(NW, NB, KB)
    src4 = src3.reshape(NW, NCH, CNB, KB)
    dst4 = dst3.reshape(NW, NCH, CNB, KB)

    ones = jnp.ones((KB, D), jnp.float32)
    zeros = jnp.zeros((N, D), jnp.float32)

    hh3 = head_idx.reshape(NW, TNB, TKB)
    rr3 = rel_idx.reshape(NW, TNB, TKB)
    tt3 = tail_idx.reshape(NW, TNB, TKB)
    idx3 = jnp.concatenate([hh3, rr3, tt3], axis=1)

    degp = _sc_degree(dst3, ones, zeros)

    h1 = _tc_mm1(x, W1)
    g1 = _tc_scale(h1, degp)
    acc1 = _sc_edge_scatter(g1, src4, dst4, zeros)
    g2 = _tc_mid(acc1, g1, degp, b1.reshape(1, D), W2)
    acc2 = _sc_edge_scatter(g2, src4, dst4, zeros)
    h = _tc_post2(acc2, g2, degp, b2.reshape(1, D))

    hh, hr, ht = _sc_triple_gather(h, rel_emb, idx3)

    Wm2p = jnp.zeros((D, D), jnp.float32).at[:, :3].set(Wm2)
    bm2p = jnp.zeros((1, D), jnp.float32).at[0, :3].set(bm2)
    out = _tc_mlp(hh, hr, ht, Wm1, bm1.reshape(1, D), Wm2p, bm2p)
    return out[:, :3]
